# Initial kernel scaffold; baseline (speedup 1.0000x reference)
import jax
import jax.numpy as jnp
from jax.experimental import pallas as pl


def kernel(latents, embedding_weight):
    raise NotImplementedError("write your pallas kernel here")


# speedup vs baseline: 1.3411x; 1.3411x over previous
import functools

import jax
import jax.numpy as jnp
from jax import lax
from jax.experimental import pallas as pl
from jax.experimental.pallas import tpu as pltpu
from jax.experimental.pallas import tpu_sc as plsc

K = 8192
D = 256
N = 8192
BETA = 0.25

BM = 256
NBLK = N // BM


def _half_argmin(dist, base):
    rowmin = jnp.min(dist, axis=1, keepdims=True)
    cols = lax.broadcasted_iota(jnp.int32, dist.shape, 1) + base
    idx = jnp.min(jnp.where(dist == rowmin, cols, K), axis=1)
    return rowmin[:, 0], idx


def _argmin_body(x_ref, w_ref, xsq_ref, wsq_ref, inds_ref, minval_ref):
    x = x_ref[...]
    w = w_ref[...]
    mm = lax.dot_general(x, w, (((1,), (1,)), ((), ())),
                         preferred_element_type=jnp.float32)
    t1 = xsq_ref[0] + wsq_ref[...]
    dist = t1 - 2.0 * mm
    m1, i1 = _half_argmin(dist[:, : K // 2], 0)
    m2, i2 = _half_argmin(dist[:, K // 2 :], K // 2)
    carry = m1.astype(jnp.bfloat16).astype(jnp.float32)
    win2 = m2 < carry
    inds_ref[0, 0, :] = jnp.where(win2, i2, i1)
    minval_ref[0, 0, :] = jnp.where(win2, m2, m1)


def _argmin_call(flat, w, xsq3, wsq2):
    return pl.pallas_call(
        _argmin_body,
        grid=(NBLK,),
        in_specs=[
            pl.BlockSpec((BM, D), lambda i: (i, 0)),
            pl.BlockSpec((K, D), lambda i: (0, 0)),
            pl.BlockSpec((1, BM, 1), lambda i: (i, 0, 0)),
            pl.BlockSpec((1, K), lambda i: (0, 0)),
        ],
        out_specs=[
            pl.BlockSpec((1, 1, BM), lambda i: (i, 0, 0)),
            pl.BlockSpec((1, 1, BM), lambda i: (i, 0, 0)),
        ],
        out_shape=[
            jax.ShapeDtypeStruct((NBLK, 1, BM), jnp.int32),
            jax.ShapeDtypeStruct((NBLK, 1, BM), jnp.float32),
        ],
    )(flat, w, xsq3, wsq2)


_NC, _NS = 2, 16
_NW = _NC * _NS
_RPW = N // _NW
_CHUNK = 128
_NCHUNK = _RPW // _CHUNK


def _sc_gather_hist(w_hbm, idx_hbm, out_hbm, cnt_hbm, idx_v, rows_v, cnt_v, sem):
    wid = lax.axis_index("s") * _NC + lax.axis_index("c")
    base = wid * _RPW
    pltpu.sync_copy(idx_hbm.at[pl.ds(_NCHUNK * wid, _NCHUNK)], idx_v)

    def _zero(i, _):
        cnt_v[pl.ds(i * 16, 16)] = jnp.zeros((16,), jnp.float32)
        return _
    lax.fori_loop(0, K // 16, _zero, 0)

    ones = jnp.ones((16,), jnp.float32)
    for c in range(_NCHUNK):
        pltpu.async_copy(w_hbm.at[idx_v.at[c]], rows_v, sem).wait()
        pltpu.sync_copy(rows_v, out_hbm.at[pl.ds(base + c * _CHUNK, _CHUNK)])

        def _hist(j, _):
            vec = idx_v[c, pl.ds(j * 16, 16)]
            plsc.addupdate_scatter(cnt_v, [vec], ones)
            return _
        lax.fori_loop(0, _CHUNK // 16, _hist, 0)

    pltpu.sync_copy(cnt_v, cnt_hbm.at[wid])


def _sc_call(w, idx2):
    f = pl.kernel(
        _sc_gather_hist,
        mesh=plsc.VectorSubcoreMesh(core_axis_name="c", subcore_axis_name="s"),
        out_type=[
            jax.ShapeDtypeStruct((N, D), jnp.float32),
            jax.ShapeDtypeStruct((_NW, K), jnp.float32),
        ],
        scratch_types=[
            pltpu.VMEM((_NCHUNK, _CHUNK), jnp.int32),
            pltpu.VMEM((_CHUNK, D), jnp.float32),
            pltpu.VMEM((K,), jnp.float32),
            pltpu.SemaphoreType.DMA,
        ],
        compiler_params=pltpu.CompilerParams(needs_layout_passes=False),
    )
    return f(w, idx2)


def _scalars_body(minval_ref, cnt_ref, loss_ref, perp_ref):
    sse = jnp.sum(minval_ref[...])
    loss_ref[...] = (sse * (1.25 / (N * D))).reshape(1, 1)
    counts = jnp.sum(cnt_ref[...], axis=0)
    p = counts * (1.0 / N)
    ent = jnp.sum(p * jnp.log(p + 1e-10))
    perp_ref[...] = jnp.exp(-ent).reshape(1, 1)


def _scalars_call(minval2, cnt):
    return pl.pallas_call(
        _scalars_body,
        in_specs=[
            pl.BlockSpec((N // 128, 128), lambda: (0, 0)),
            pl.BlockSpec((_NW, K), lambda: (0, 0)),
        ],
        out_specs=[
            pl.BlockSpec((1, 1), lambda: (0, 0)),
            pl.BlockSpec((1, 1), lambda: (0, 0)),
        ],
        out_shape=[
            jax.ShapeDtypeStruct((1, 1), jnp.float32),
            jax.ShapeDtypeStruct((1, 1), jnp.float32),
        ],
    )(minval2, cnt)


def kernel(latents, embedding_weight):
    flat = latents.reshape(-1, D)
    xsq = jnp.sum(flat ** 2, axis=1)
    wsq = jnp.sum(embedding_weight ** 2, axis=1)

    inds3, minval3 = _argmin_call(
        flat.astype(jnp.bfloat16), embedding_weight.astype(jnp.bfloat16),
        xsq.reshape(NBLK, BM, 1), wsq.reshape(1, K))

    quant_flat, cnt = _sc_call(
        embedding_weight, inds3.reshape(N // _CHUNK, _CHUNK))

    vq_loss, perp = _scalars_call(minval3.reshape(N // 128, 128), cnt)

    return (quant_flat.reshape(latents.shape), vq_loss[0, 0], perp[0, 0])


# speedup vs baseline: 1.3720x; 1.0231x over previous
import functools

import jax
import jax.numpy as jnp
from jax import lax
from jax.experimental import pallas as pl
from jax.experimental.pallas import tpu as pltpu
from jax.experimental.pallas import tpu_sc as plsc

K = 8192
D = 256
N = 8192
BETA = 0.25

BM = 512
NBLK = N // BM


def _half_argmin(dist, base):
    rowmin = jnp.min(dist, axis=1, keepdims=True)
    cols = lax.broadcasted_iota(jnp.int32, dist.shape, 1) + base
    idx = jnp.min(jnp.where(dist == rowmin, cols, K), axis=1)
    return rowmin[:, 0], idx


def _argmin_body(x_ref, w_ref, xsq_ref, wsq_ref, inds_ref, minval_ref):
    x = x_ref[...]
    w = w_ref[...]
    mm = lax.dot_general(x, w, (((1,), (1,)), ((), ())),
                         preferred_element_type=jnp.float32)
    t1 = xsq_ref[0] + wsq_ref[...]
    dist = t1 - 2.0 * mm
    m1, i1 = _half_argmin(dist[:, : K // 2], 0)
    m2, i2 = _half_argmin(dist[:, K // 2 :], K // 2)
    carry = m1.astype(jnp.bfloat16).astype(jnp.float32)
    win2 = m2 < carry
    inds_ref[0, 0, :] = jnp.where(win2, i2, i1)
    minval_ref[0, 0, :] = jnp.where(win2, m2, m1)


def _argmin_call(flat, w, xsq3, wsq2):
    return pl.pallas_call(
        _argmin_body,
        grid=(NBLK,),
        in_specs=[
            pl.BlockSpec((BM, D), lambda i: (i, 0)),
            pl.BlockSpec((K, D), lambda i: (0, 0)),
            pl.BlockSpec((1, BM, 1), lambda i: (i, 0, 0)),
            pl.BlockSpec((1, K), lambda i: (0, 0)),
        ],
        out_specs=[
            pl.BlockSpec((1, 1, BM), lambda i: (i, 0, 0)),
            pl.BlockSpec((1, 1, BM), lambda i: (i, 0, 0)),
        ],
        out_shape=[
            jax.ShapeDtypeStruct((NBLK, 1, BM), jnp.int32),
            jax.ShapeDtypeStruct((NBLK, 1, BM), jnp.float32),
        ],
    )(flat, w, xsq3, wsq2)


_NC, _NS = 2, 16
_NW = _NC * _NS
_RPW = N // _NW
_CHUNK = 128
_NCHUNK = _RPW // _CHUNK


def _sc_gather_hist(w_hbm, idx_hbm, out_hbm, cnt_hbm, idx_v, rows_v, cnt_v, sem):
    wid = lax.axis_index("s") * _NC + lax.axis_index("c")
    base = wid * _RPW
    pltpu.sync_copy(idx_hbm.at[pl.ds(_NCHUNK * wid, _NCHUNK)], idx_v)

    def _zero(i, _):
        cnt_v[pl.ds(i * 16, 16)] = jnp.zeros((16,), jnp.float32)
        return _
    lax.fori_loop(0, K // 16, _zero, 0)

    ones = jnp.ones((16,), jnp.float32)
    for c in range(_NCHUNK):
        pltpu.async_copy(w_hbm.at[idx_v.at[c]], rows_v, sem).wait()
        pltpu.sync_copy(rows_v, out_hbm.at[pl.ds(base + c * _CHUNK, _CHUNK)])

        def _hist(j, _):
            vec = idx_v[c, pl.ds(j * 16, 16)]
            plsc.addupdate_scatter(cnt_v, [vec], ones)
            return _
        lax.fori_loop(0, _CHUNK // 16, _hist, 0)

    pltpu.sync_copy(cnt_v, cnt_hbm.at[wid])


def _sc_call(w, idx2):
    f = pl.kernel(
        _sc_gather_hist,
        mesh=plsc.VectorSubcoreMesh(core_axis_name="c", subcore_axis_name="s"),
        out_type=[
            jax.ShapeDtypeStruct((N, D), jnp.float32),
            jax.ShapeDtypeStruct((_NW, K), jnp.float32),
        ],
        scratch_types=[
            pltpu.VMEM((_NCHUNK, _CHUNK), jnp.int32),
            pltpu.VMEM((_CHUNK, D), jnp.float32),
            pltpu.VMEM((K,), jnp.float32),
            pltpu.SemaphoreType.DMA,
        ],
        compiler_params=pltpu.CompilerParams(needs_layout_passes=False),
    )
    return f(w, idx2)


def _scalars_body(minval_ref, cnt_ref, loss_ref, perp_ref):
    sse = jnp.sum(minval_ref[...])
    loss_ref[...] = (sse * (1.25 / (N * D))).reshape(1, 1)
    counts = jnp.sum(cnt_ref[...], axis=0)
    p = counts * (1.0 / N)
    ent = jnp.sum(p * jnp.log(p + 1e-10))
    perp_ref[...] = jnp.exp(-ent).reshape(1, 1)


def _scalars_call(minval2, cnt):
    return pl.pallas_call(
        _scalars_body,
        in_specs=[
            pl.BlockSpec((N // 128, 128), lambda: (0, 0)),
            pl.BlockSpec((_NW, K), lambda: (0, 0)),
        ],
        out_specs=[
            pl.BlockSpec((1, 1), lambda: (0, 0)),
            pl.BlockSpec((1, 1), lambda: (0, 0)),
        ],
        out_shape=[
            jax.ShapeDtypeStruct((1, 1), jnp.float32),
            jax.ShapeDtypeStruct((1, 1), jnp.float32),
        ],
    )(minval2, cnt)


def kernel(latents, embedding_weight):
    flat = latents.reshape(-1, D)
    xsq = jnp.sum(flat ** 2, axis=1)
    wsq = jnp.sum(embedding_weight ** 2, axis=1)

    inds3, minval3 = _argmin_call(
        flat.astype(jnp.bfloat16), embedding_weight.astype(jnp.bfloat16),
        xsq.reshape(NBLK, BM, 1), wsq.reshape(1, K))

    quant_flat, cnt = _sc_call(
        embedding_weight, inds3.reshape(N // _CHUNK, _CHUNK))

    vq_loss, perp = _scalars_call(minval3.reshape(N // 128, 128), cnt)

    return (quant_flat.reshape(latents.shape), vq_loss[0, 0], perp[0, 0])
